# Initial kernel scaffold; baseline (speedup 1.0000x reference)
import jax
import jax.numpy as jnp
from jax.experimental import pallas as pl


def kernel(node_features, edge_index, edge_attr, edge_weights, W1, b1, W2, b2, W3, b3, ln_g, ln_b, W4, b4):
    raise NotImplementedError("write your pallas kernel here")


# speedup vs baseline: 2.5385x; 2.5385x over previous
import functools

import jax
import jax.numpy as jnp
from jax import lax
from jax.experimental import pallas as pl
from jax.experimental.pallas import tpu as pltpu
from jax.experimental.pallas import tpu_sc as plsc

N_NODES = 10000
N_EDGES = 320000
NODE_DIM = 128
EDGE_DIM = 16
HIDDEN = 64
AW = 80

NC, NS = 2, 16
NW = NC * NS
EPW = N_EDGES // NW
CHUNK = 80
NCHUNK = EPW // CHUNK
NSTAGE = 10
RPS = N_NODES // NSTAGE


def _p_body(x_ref, w1_ref, p_ref):
    p_ref[...] = jnp.dot(x_ref[...], w1_ref[:NODE_DIM],
                         preferred_element_type=jnp.float32)


def _compute_p(x, w1):
    return pl.pallas_call(
        _p_body,
        out_shape=jax.ShapeDtypeStruct((N_NODES, HIDDEN), jnp.float32),
    )(x, w1)


_QB = 10000


def _q_body(e_ref, w1_ref, b1_ref, q_ref):
    q_ref[...] = (jnp.dot(e_ref[...], w1_ref[NODE_DIM:],
                          preferred_element_type=jnp.float32)
                  + b1_ref[...])


def _compute_q(e, w1, b1):
    return pl.pallas_call(
        _q_body,
        grid=(N_EDGES // _QB,),
        in_specs=[
            pl.BlockSpec((_QB, EDGE_DIM), lambda i: (i, 0)),
            pl.BlockSpec((NODE_DIM + EDGE_DIM, HIDDEN), lambda i: (0, 0)),
            pl.BlockSpec((1, HIDDEN), lambda i: (0, 0)),
        ],
        out_specs=pl.BlockSpec((_QB, HIDDEN), lambda i: (i, 0)),
        out_shape=jax.ShapeDtypeStruct((N_EDGES, HIDDEN), jnp.float32),
    )(e, w1, b1.reshape(1, HIDDEN))


def _sc_edge_body(p_hbm, q_hbm, w_hbm, src_hbm, dst_hbm, z_hbm, a_out,
                  src_buf, dst_buf, w_buf, q_buf, p_buf, m_buf, a_sh, sem):
    c = lax.axis_index("c")
    s = lax.axis_index("s")
    wid = s * NC + c
    r0 = pl.multiple_of(s * RPS, 8)

    @pl.when(s < NSTAGE)
    def _stage():
        pltpu.sync_copy(z_hbm.at[pl.ds(r0, RPS), :], a_sh.at[pl.ds(r0, RPS), :])

    plsc.subcore_barrier()

    def chunk(ci, carry):
        base = pl.multiple_of(wid * EPW + ci * CHUNK, 8)
        pltpu.sync_copy(src_hbm.at[pl.ds(base, CHUNK)], src_buf)
        pltpu.sync_copy(dst_hbm.at[pl.ds(base, CHUNK)], dst_buf)
        pltpu.sync_copy(w_hbm.at[pl.ds(base, CHUNK)], w_buf)
        pltpu.sync_copy(q_hbm.at[pl.ds(base, CHUNK), :], q_buf)
        pltpu.async_copy(p_hbm.at[src_buf], p_buf, sem).wait()
        for grp in range(CHUNK // 16):
            w16 = w_buf[pl.ds(grp * 16, 16)]
            for j in range(16):
                g = grp * 16 + j
                wj = w16[j]
                for k in range(HIDDEN // 16):
                    z = p_buf[g, pl.ds(16 * k, 16)] + q_buf[g, pl.ds(16 * k, 16)]
                    h = jnp.maximum(z, 0.1 * z)
                    m_buf[g, pl.ds(16 * k, 16)] = wj * h
                m_buf[g, pl.ds(HIDDEN, 16)] = jnp.full((16,), wj, jnp.float32)
        pltpu.sync_copy(m_buf, a_sh.at[dst_buf], add=True)
        return carry

    lax.fori_loop(0, NCHUNK, chunk, 0)
    plsc.subcore_barrier()

    @pl.when(s < NSTAGE)
    def _drain():
        pltpu.sync_copy(a_sh.at[pl.ds(r0, RPS), :],
                        a_out.at[c, pl.ds(r0, RPS), :])


def _sc_edge(p, q, w, src, dst):
    zeros = jnp.zeros((N_NODES, AW), jnp.float32)
    mesh = plsc.VectorSubcoreMesh(core_axis_name="c", subcore_axis_name="s")
    f = pl.kernel(
        _sc_edge_body,
        out_type=jax.ShapeDtypeStruct((NC, N_NODES, AW), jnp.float32),
        mesh=mesh,
        scratch_types=[
            pltpu.VMEM((CHUNK,), jnp.int32),
            pltpu.VMEM((CHUNK,), jnp.int32),
            pltpu.VMEM((CHUNK,), jnp.float32),
            pltpu.VMEM((CHUNK, HIDDEN), jnp.float32),
            pltpu.VMEM((CHUNK, HIDDEN), jnp.float32),
            pltpu.VMEM((CHUNK, AW), jnp.float32),
            pltpu.VMEM_SHARED((N_NODES, AW), jnp.float32),
            pltpu.SemaphoreType.DMA,
        ],
        compiler_params=pltpu.CompilerParams(use_tc_tiling_on_sc=False),
    )
    return f(p, q, w, src, dst, zeros)


def _post_body(x_ref, a_ref, w2_ref, b2_ref, w3_ref, b3_ref, g_ref, be_ref,
               w4_ref, b4_ref, o_ref):
    x = x_ref[...]
    a = a_ref[0] + a_ref[1]
    a64 = a[:, :HIDDEN]
    s16 = a[:, HIDDEN:]
    w3x = w3_ref[:NODE_DIM]
    w3a = w3_ref[NODE_DIM:]
    m = jnp.dot(w2_ref[...], w3a, preferred_element_type=jnp.float32)
    v = jnp.dot(b2_ref[...], w3a, preferred_element_type=jnp.float32)
    v_ext = jnp.concatenate([v, jnp.zeros((15, HIDDEN), jnp.float32)], axis=0)
    u = (jnp.dot(x, w3x, preferred_element_type=jnp.float32)
         + jnp.dot(a64, m, preferred_element_type=jnp.float32)
         + jnp.dot(s16, v_ext, preferred_element_type=jnp.float32)
         + b3_ref[...])
    mean = jnp.mean(u, axis=-1, keepdims=True)
    var = jnp.mean((u - mean) ** 2, axis=-1, keepdims=True)
    u = (u - mean) / jnp.sqrt(var + 1e-5) * g_ref[...] + be_ref[...]
    u = jnp.maximum(u, 0.1 * u)
    o_ref[...] = jnp.dot(u, w4_ref[...], preferred_element_type=jnp.float32) + b4_ref[...]


def _post(x, a_ext, w2, b2, w3, b3, ln_g, ln_b, w4, b4):
    return pl.pallas_call(
        _post_body,
        out_shape=jax.ShapeDtypeStruct((N_NODES, NODE_DIM), jnp.float32),
    )(x, a_ext, w2, b2.reshape(1, NODE_DIM), w3, b3.reshape(1, HIDDEN),
      ln_g.reshape(1, HIDDEN), ln_b.reshape(1, HIDDEN), w4,
      b4.reshape(1, NODE_DIM))


def kernel(node_features, edge_index, edge_attr, edge_weights,
           W1, b1, W2, b2, W3, b3, ln_g, ln_b, W4, b4):
    src = edge_index[0].astype(jnp.int32)
    dst = edge_index[1].astype(jnp.int32)
    p = _compute_p(node_features, W1)
    q = _compute_q(edge_attr, W1, b1)
    a_ext = _sc_edge(p, q, edge_weights, src, dst)
    return _post(node_features, a_ext, W2, b2, W3, b3, ln_g, ln_b, W4, b4)
